# baseline (speedup 1.0000x reference)
import functools

import jax
import jax.numpy as jnp
from jax import lax
from jax.experimental import pallas as pl
from jax.experimental.pallas import tpu as pltpu
from jax.experimental.pallas import tpu_sc as plsc

_V = 1000000
_D = 32
_B = 4096
_L = 200

_NC = 2
_NS = 16
_NW = _NC * _NS
_BPW = _B // _NW
_C0 = 128
_C1 = _L - _C0
_LANES = 16

_NSLOTS = 8
_SCALE = 2048


def _body(x_hbm, lens_hbm, table_hbm, out_hbm, xblk_v, lens_v, buf_v, out_v, sems):
    wid = lax.axis_index("s") * _NC + lax.axis_index("c")
    base = wid * _BPW

    pltpu.sync_copy(x_hbm.at[pl.ds(base, _BPW)], xblk_v)
    pltpu.sync_copy(lens_hbm.at[pl.ds(base * _LANES, _BPW * _LANES)], lens_v)

    _CHUNKS = ((0, _C0), (_C0, _C1))

    def gather_row(b, slot):
        return tuple(
            pltpu.make_async_copy(
                table_hbm.at[xblk_v.at[b, pl.ds(off, n)]],
                buf_v.at[slot, pl.ds(off, n)],
                sems.at[slot],
            )
            for off, n in _CHUNKS
        )

    def fire(b, slot):
        for h in gather_row(b, slot):
            h.start()

    def drain(b, slot):
        for h in gather_row(b, slot):
            h.wait()

    def accumulate(b, slot):
        def acc_body(i, carry):
            a0, a1 = carry
            t0 = i * 8
            for j in range(8):
                w = buf_v[slot, t0 + j, pl.ds(0, _LANES)]
                a0 = a0 + (w & 65535)
                a1 = a1 + (w >> 16)
            return (a0, a1)

        zero = jnp.zeros((_LANES,), jnp.int32)
        a0, a1 = lax.fori_loop(0, _L // 8, acc_body, (zero, zero))
        f0 = (a0 - _L * 32768).astype(jnp.float32)
        f1 = a1.astype(jnp.float32)
        linv = 1.0 / (lens_v[pl.ds(b * _LANES, _LANES)] * float(_SCALE))
        out_v[pl.ds(b * _D, _LANES)] = f0 * linv
        out_v[pl.ds(b * _D + _LANES, _LANES)] = f1 * linv

    for s in range(_NSLOTS):
        fire(s, s)

    def group_body(g, _):
        r0 = g * _NSLOTS
        for s in range(_NSLOTS):
            drain(r0 + s, s)
            accumulate(r0 + s, s)

            @pl.when(r0 + s + _NSLOTS < _BPW)
            def _():
                fire(r0 + s + _NSLOTS, s)

        return 0

    lax.fori_loop(0, _BPW // _NSLOTS, group_body, 0)

    pltpu.sync_copy(out_v, out_hbm.at[pl.ds(base * _D, _BPW * _D)])


_wordvec_avg = functools.partial(
    pl.kernel,
    out_type=jax.ShapeDtypeStruct((_B * _D,), jnp.float32),
    mesh=plsc.VectorSubcoreMesh(
        core_axis_name="c", subcore_axis_name="s", num_cores=_NC, num_subcores=_NS
    ),
    scratch_types=[
        pltpu.VMEM((_BPW, _L), jnp.int32),
        pltpu.VMEM((_BPW * _LANES,), jnp.float32),
        pltpu.VMEM((_NSLOTS, _L, _LANES), jnp.int32),
        pltpu.VMEM((_BPW * _D,), jnp.float32),
        pltpu.SemaphoreType.DMA((_NSLOTS,)),
    ],
    compiler_params=pltpu.CompilerParams(use_tc_tiling_on_sc=False),
)(_body)


def kernel(x, x_lens, table):
    lens_rep = jnp.repeat(x_lens, _LANES)
    q = jnp.clip(jnp.round(table * float(_SCALE)), -32768, 32767).astype(jnp.int32)
    packed = (q[:, :_LANES] + 32768) | (q[:, _LANES:] << 16)
    return _wordvec_avg(x, lens_rep, packed).reshape(_B, _D)


# speedup vs baseline: 2.3902x; 2.3902x over previous
import functools

import jax
import jax.numpy as jnp
from jax import lax
from jax.experimental import pallas as pl
from jax.experimental.pallas import tpu as pltpu
from jax.experimental.pallas import tpu_sc as plsc

_V = 1000000
_D = 32
_B = 4096
_L = 200

_NC = 2
_NS = 16
_NW = _NC * _NS
_BPW = _B // _NW
_C0 = 128
_C1 = _L - _C0
_LANES = 16

_NSLOTS = 8


def _body(x_hbm, lens_hbm, table_hbm, out_hbm, xblk_v, lens_v, buf_v, out_v, sems):
    wid = lax.axis_index("s") * _NC + lax.axis_index("c")
    base = wid * _BPW

    pltpu.sync_copy(x_hbm.at[pl.ds(base, _BPW)], xblk_v)
    pltpu.sync_copy(lens_hbm.at[pl.ds(base * _LANES, _BPW * _LANES)], lens_v)

    _CHUNKS = ((0, _C0), (_C0, _C1))

    def gather_row(b, slot):
        return tuple(
            pltpu.make_async_copy(
                table_hbm.at[xblk_v.at[b, pl.ds(off, n)]],
                buf_v.at[slot, pl.ds(off, n)],
                sems.at[slot],
            )
            for off, n in _CHUNKS
        )

    def fire(b, slot):
        for h in gather_row(b, slot):
            h.start()

    def drain(b, slot):
        for h in gather_row(b, slot):
            h.wait()

    def accumulate(b, slot):
        def acc_body(i, carry):
            a0, a1 = carry
            t0 = i * 8
            for j in range(8):
                a0 = a0 + buf_v[slot, t0 + j, pl.ds(0, _LANES)]
                a1 = a1 + buf_v[slot, t0 + j, pl.ds(_LANES, _LANES)]
            return (a0, a1)

        zero = jnp.zeros((_LANES,), jnp.float32)
        a0, a1 = lax.fori_loop(0, _L // 8, acc_body, (zero, zero))
        linv = 1.0 / lens_v[pl.ds(b * _LANES, _LANES)]
        out_v[pl.ds(b * _D, _LANES)] = a0 * linv
        out_v[pl.ds(b * _D + _LANES, _LANES)] = a1 * linv

    for s in range(_NSLOTS):
        fire(s, s)

    def group_body(g, _):
        r0 = g * _NSLOTS
        for s in range(_NSLOTS):
            drain(r0 + s, s)
            accumulate(r0 + s, s)

            @pl.when(r0 + s + _NSLOTS < _BPW)
            def _():
                fire(r0 + s + _NSLOTS, s)

        return 0

    lax.fori_loop(0, _BPW // _NSLOTS, group_body, 0)

    pltpu.sync_copy(out_v, out_hbm.at[pl.ds(base * _D, _BPW * _D)])


_wordvec_avg = functools.partial(
    pl.kernel,
    out_type=jax.ShapeDtypeStruct((_B * _D,), jnp.float32),
    mesh=plsc.VectorSubcoreMesh(
        core_axis_name="c", subcore_axis_name="s", num_cores=_NC, num_subcores=_NS
    ),
    scratch_types=[
        pltpu.VMEM((_BPW, _L), jnp.int32),
        pltpu.VMEM((_BPW * _LANES,), jnp.float32),
        pltpu.VMEM((_NSLOTS, _L, _D), jnp.float32),
        pltpu.VMEM((_BPW * _D,), jnp.float32),
        pltpu.SemaphoreType.DMA((_NSLOTS,)),
    ],
    compiler_params=pltpu.CompilerParams(use_tc_tiling_on_sc=False),
)(_body)


def kernel(x, x_lens, table):
    lens_rep = jnp.repeat(x_lens, _LANES)
    return _wordvec_avg(x, lens_rep, table).reshape(_B, _D)


# speedup vs baseline: 2.3908x; 1.0003x over previous
import functools

import jax
import jax.numpy as jnp
from jax import lax
from jax.experimental import pallas as pl
from jax.experimental.pallas import tpu as pltpu
from jax.experimental.pallas import tpu_sc as plsc

_V = 1000000
_D = 32
_B = 4096
_L = 200

_NC = 2
_NS = 16
_NW = _NC * _NS
_BPW = _B // _NW
_C0 = 128
_C1 = _L - _C0
_LANES = 16

_NSLOTS = 8


def _body(x_hbm, lens_hbm, table_hbm, out_hbm, xblk_v, lens_v, buf_v, out_v, sems):
    wid = lax.axis_index("s") * _NC + lax.axis_index("c")
    base = wid * _BPW

    pltpu.sync_copy(x_hbm.at[pl.ds(base, _BPW)], xblk_v)
    pltpu.sync_copy(lens_hbm.at[pl.ds(base * _LANES, _BPW * _LANES)], lens_v)

    _CHUNKS = ((0, _L),)

    def gather_row(b, slot):
        return tuple(
            pltpu.make_async_copy(
                table_hbm.at[xblk_v.at[b, pl.ds(off, n)]],
                buf_v.at[slot, pl.ds(off, n)],
                sems.at[slot],
            )
            for off, n in _CHUNKS
        )

    def fire(b, slot):
        for h in gather_row(b, slot):
            h.start()

    def drain(b, slot):
        for h in gather_row(b, slot):
            h.wait()

    def accumulate(b, slot):
        def acc_body(i, carry):
            a0, a1 = carry
            t0 = i * 8
            for j in range(8):
                a0 = a0 + buf_v[slot, t0 + j, pl.ds(0, _LANES)]
                a1 = a1 + buf_v[slot, t0 + j, pl.ds(_LANES, _LANES)]
            return (a0, a1)

        zero = jnp.zeros((_LANES,), jnp.float32)
        a0, a1 = lax.fori_loop(0, _L // 8, acc_body, (zero, zero))
        linv = 1.0 / lens_v[pl.ds(b * _LANES, _LANES)]
        out_v[pl.ds(b * _D, _LANES)] = a0 * linv
        out_v[pl.ds(b * _D + _LANES, _LANES)] = a1 * linv

    for s in range(_NSLOTS):
        fire(s, s)

    def group_body(g, _):
        r0 = g * _NSLOTS
        for s in range(_NSLOTS):
            drain(r0 + s, s)
            accumulate(r0 + s, s)

            @pl.when(r0 + s + _NSLOTS < _BPW)
            def _():
                fire(r0 + s + _NSLOTS, s)

        return 0

    lax.fori_loop(0, _BPW // _NSLOTS, group_body, 0)

    pltpu.sync_copy(out_v, out_hbm.at[pl.ds(base * _D, _BPW * _D)])


_wordvec_avg = functools.partial(
    pl.kernel,
    out_type=jax.ShapeDtypeStruct((_B * _D,), jnp.float32),
    mesh=plsc.VectorSubcoreMesh(
        core_axis_name="c", subcore_axis_name="s", num_cores=_NC, num_subcores=_NS
    ),
    scratch_types=[
        pltpu.VMEM((_BPW, _L), jnp.int32),
        pltpu.VMEM((_BPW * _LANES,), jnp.float32),
        pltpu.VMEM((_NSLOTS, _L, _D), jnp.float32),
        pltpu.VMEM((_BPW * _D,), jnp.float32),
        pltpu.SemaphoreType.DMA((_NSLOTS,)),
    ],
    compiler_params=pltpu.CompilerParams(use_tc_tiling_on_sc=False),
)(_body)


def kernel(x, x_lens, table):
    lens_rep = jnp.repeat(x_lens, _LANES)
    return _wordvec_avg(x, lens_rep, table).reshape(_B, _D)


# speedup vs baseline: 2.3930x; 1.0009x over previous
import functools

import jax
import jax.numpy as jnp
from jax import lax
from jax.experimental import pallas as pl
from jax.experimental.pallas import tpu as pltpu
from jax.experimental.pallas import tpu_sc as plsc

_V = 1000000
_D = 32
_B = 4096
_L = 200

_NC = 2
_NS = 16
_NW = _NC * _NS
_BPW = _B // _NW
_LANES = 16

_NSLOTS = 8


def _body(x_hbm, lens_hbm, table_hbm, out_hbm, xblk_v, lens_v, buf_v, out_v, sems):
    wid = lax.axis_index("s") * _NC + lax.axis_index("c")
    base = wid * _BPW

    pltpu.sync_copy(x_hbm.at[pl.ds(base, _BPW)], xblk_v)
    pltpu.sync_copy(lens_hbm.at[pl.ds(base * _LANES, _BPW * _LANES)], lens_v)

    _CHUNKS = ((0, _L),)

    def gather_row(b, slot):
        return tuple(
            pltpu.make_async_copy(
                table_hbm.at[xblk_v.at[b, pl.ds(off, n)]],
                buf_v.at[slot, pl.ds(off, n)],
                sems.at[slot],
            )
            for off, n in _CHUNKS
        )

    def fire(b, slot):
        for h in gather_row(b, slot):
            h.start()

    def drain(b, slot):
        for h in gather_row(b, slot):
            h.wait()

    def accumulate(b, slot):
        def acc_body(i, carry):
            a0, a1 = carry
            t0 = i * 8
            for j in range(8):
                a0 = a0 + buf_v[slot, t0 + j, pl.ds(0, _LANES)]
                a1 = a1 + buf_v[slot, t0 + j, pl.ds(_LANES, _LANES)]
            return (a0, a1)

        zero = jnp.zeros((_LANES,), jnp.float32)
        a0, a1 = lax.fori_loop(0, _L // 8, acc_body, (zero, zero))
        linv = 1.0 / lens_v[pl.ds(b * _LANES, _LANES)]
        out_v[pl.ds(b * _D, _LANES)] = a0 * linv
        out_v[pl.ds(b * _D + _LANES, _LANES)] = a1 * linv

    for s in range(_NSLOTS):
        fire(s, s)

    def group_body(g, _):
        r0 = g * _NSLOTS
        for s in range(_NSLOTS):
            drain(r0 + s, s)
            accumulate(r0 + s, s)

            @pl.when(r0 + s + _NSLOTS < _BPW)
            def _():
                fire(r0 + s + _NSLOTS, s)

        return 0

    lax.fori_loop(0, _BPW // _NSLOTS, group_body, 0)

    pltpu.sync_copy(out_v, out_hbm.at[pl.ds(base * _D, _BPW * _D)])


_wordvec_avg = functools.partial(
    pl.kernel,
    out_type=jax.ShapeDtypeStruct((_B * _D,), jnp.float32),
    mesh=plsc.VectorSubcoreMesh(
        core_axis_name="c", subcore_axis_name="s", num_cores=_NC, num_subcores=_NS
    ),
    scratch_types=[
        pltpu.VMEM((_BPW, _L), jnp.int32),
        pltpu.VMEM((_BPW * _LANES,), jnp.float32),
        pltpu.VMEM((_NSLOTS, _L, _D), jnp.float32),
        pltpu.VMEM((_BPW * _D,), jnp.float32),
        pltpu.SemaphoreType.DMA((_NSLOTS,)),
    ],
    compiler_params=pltpu.CompilerParams(use_tc_tiling_on_sc=False),
)(_body)


def kernel(x, x_lens, table):
    lens_rep = jnp.repeat(x_lens, _LANES)
    return _wordvec_avg(x, lens_rep, table).reshape(_B, _D)
